# baseline (speedup 1.0000x reference)
import functools

import jax
import jax.numpy as jnp
from jax import lax
from jax.experimental import pallas as pl
from jax.experimental.pallas import tpu as pltpu
from jax.experimental.pallas import tpu_sc as plsc

N = 10000
NP = 10240
D = 128
E = 320000
NC = 2
NS = 16
NW = NC * NS
CH = 128
NCH = 80
EP = NW * NCH * CH
RPT = NP // NS

_mesh = plsc.VectorSubcoreMesh(core_axis_name="c", subcore_axis_name="s")


@functools.partial(
    pl.kernel,
    out_type=(
        jax.ShapeDtypeStruct((NC * NP,), jnp.float32),
        jax.ShapeDtypeStruct((NC * NP,), jnp.float32),
    ),
    mesh=_mesh,
    scratch_types=[
        pltpu.VMEM((NCH, CH), jnp.int32),
        pltpu.VMEM((NCH, CH), jnp.int32),
        pltpu.VMEM((CH,), jnp.float32),
        pltpu.VMEM((RPT,), jnp.float32),
        pltpu.VMEM_SHARED((NP,), jnp.float32),
        pltpu.VMEM_SHARED((NP,), jnp.float32),
    ],
)
def _sc_degrees(src_hbm, dst_hbm, do_hbm, di_hbm,
                idx_s, idx_d, ones_v, zero_v, acc_o, acc_i):
    c = lax.axis_index("c")
    s = lax.axis_index("s")
    t = c * NS + s

    for i in range(CH // 16):
        ones_v[pl.ds(i * 16, 16)] = jnp.full((16,), 1.0, jnp.float32)
    for i in range(RPT // 16):
        zero_v[pl.ds(i * 16, 16)] = jnp.zeros((16,), jnp.float32)

    pltpu.sync_copy(zero_v, acc_o.at[pl.ds(s * RPT, RPT)])
    pltpu.sync_copy(zero_v, acc_i.at[pl.ds(s * RPT, RPT)])
    pltpu.sync_copy(src_hbm.at[t], idx_s)
    pltpu.sync_copy(dst_hbm.at[t], idx_d)
    plsc.subcore_barrier()

    def body(j, carry):
        pltpu.sync_copy(ones_v, acc_o.at[idx_s.at[j]], add=True)
        pltpu.sync_copy(ones_v, acc_i.at[idx_d.at[j]], add=True)
        return carry

    lax.fori_loop(0, NCH, body, 0)
    plsc.subcore_barrier()

    pltpu.sync_copy(acc_o.at[pl.ds(s * RPT, RPT)], do_hbm.at[pl.ds(c * NP + s * RPT, RPT)])
    pltpu.sync_copy(acc_i.at[pl.ds(s * RPT, RPT)], di_hbm.at[pl.ds(c * NP + s * RPT, RPT)])


@functools.partial(
    pl.kernel,
    out_type=jax.ShapeDtypeStruct((NC, NP, D), jnp.float32),
    mesh=_mesh,
    scratch_types=[
        pltpu.VMEM((NCH, CH), jnp.int32),
        pltpu.VMEM((NCH, CH), jnp.int32),
        pltpu.VMEM((CH, D), jnp.float32),
        pltpu.VMEM((16, D), jnp.float32),
        pltpu.VMEM_SHARED((NP, D), jnp.float32),
    ],
)
def _sc_aggregate(h_hbm, src_hbm, dst_hbm, out_hbm,
                  idx_s, idx_d, rows_v, zero_v, acc):
    c = lax.axis_index("c")
    s = lax.axis_index("s")
    t = c * NS + s

    for i in range(16):
        for cb in range(D // 16):
            zero_v[i, pl.ds(cb * 16, 16)] = jnp.zeros((16,), jnp.float32)

    def zbody(k, carry):
        pltpu.sync_copy(zero_v, acc.at[pl.ds(s * RPT + k * 16, 16)])
        return carry

    lax.fori_loop(0, RPT // 16, zbody, 0)
    pltpu.sync_copy(src_hbm.at[t], idx_s)
    pltpu.sync_copy(dst_hbm.at[t], idx_d)
    plsc.subcore_barrier()

    def body(j, carry):
        pltpu.sync_copy(h_hbm.at[idx_s.at[j]], rows_v)
        pltpu.sync_copy(rows_v, acc.at[idx_d.at[j]], add=True)
        return carry

    lax.fori_loop(0, NCH, body, 0)
    plsc.subcore_barrier()

    pltpu.sync_copy(acc.at[pl.ds(s * RPT, RPT)], out_hbm.at[c, pl.ds(s * RPT, RPT)])


RB = 2048
GRID = NP // RB
RT = RB // 128


def _norm(p):
    return lax.rsqrt(jnp.maximum(p[0] + p[1], 1.0))


def _tc1_body(feat_ref, w_ref, do_ref, out_ref):
    ns = _norm(do_ref[...])
    scaled = feat_ref[...] * ns[:, :, None]
    out_ref[...] = jnp.dot(scaled.reshape(RB, D), w_ref[...],
                           preferred_element_type=jnp.float32)


def _tc1(feat3, W1, do3):
    return pl.pallas_call(
        _tc1_body,
        grid=(GRID,),
        in_specs=[
            pl.BlockSpec((RT, 128, D), lambda i: (i, 0, 0)),
            pl.BlockSpec((D, D), lambda i: (0, 0)),
            pl.BlockSpec((NC, RT, 128), lambda i: (0, i, 0)),
        ],
        out_specs=pl.BlockSpec((RB, D), lambda i: (i, 0)),
        out_shape=jax.ShapeDtypeStruct((NP, D), jnp.float32),
    )(feat3, W1, do3)


def _tc2_body(agg_ref, di_ref, do_ref, b_ref, w_ref, out_ref):
    nd = _norm(di_ref[...])
    ns = _norm(do_ref[...])
    a = agg_ref[0] + agg_ref[1]
    h = a * nd[:, :, None] + b_ref[...]
    m = jnp.dot(h.reshape(RB, D), w_ref[...], preferred_element_type=jnp.float32)
    out_ref[...] = (m.reshape(RT, 128, D) * ns[:, :, None]).reshape(RB, D)


def _tc2(agg4, di3, do3, b1, W2):
    return pl.pallas_call(
        _tc2_body,
        grid=(GRID,),
        in_specs=[
            pl.BlockSpec((NC, RT, 128, D), lambda i: (0, i, 0, 0)),
            pl.BlockSpec((NC, RT, 128), lambda i: (0, i, 0)),
            pl.BlockSpec((NC, RT, 128), lambda i: (0, i, 0)),
            pl.BlockSpec((1, D), lambda i: (0, 0)),
            pl.BlockSpec((D, D), lambda i: (0, 0)),
        ],
        out_specs=pl.BlockSpec((RB, D), lambda i: (i, 0)),
        out_shape=jax.ShapeDtypeStruct((NP, D), jnp.float32),
    )(agg4, di3, do3, b1.reshape(1, D), W2)


def _tc3_body(agg_ref, di_ref, b_ref, feat_ref, out_ref):
    nd = _norm(di_ref[...])
    a = agg_ref[0] + agg_ref[1]
    out3 = a * nd[:, :, None] + b_ref[...] + feat_ref[...]
    out_ref[...] = out3.reshape(RB, D)


def _tc3(agg4, di3, b2, feat3):
    return pl.pallas_call(
        _tc3_body,
        grid=(GRID,),
        in_specs=[
            pl.BlockSpec((NC, RT, 128, D), lambda i: (0, i, 0, 0)),
            pl.BlockSpec((NC, RT, 128), lambda i: (0, i, 0)),
            pl.BlockSpec((1, D), lambda i: (0, 0)),
            pl.BlockSpec((RT, 128, D), lambda i: (i, 0, 0)),
        ],
        out_specs=pl.BlockSpec((RB, D), lambda i: (i, 0)),
        out_shape=jax.ShapeDtypeStruct((NP, D), jnp.float32),
    )(agg4, di3, b2.reshape(1, D), feat3)


def kernel(feat, edge_index, W1, b1, W2, b2):
    src = jnp.pad(edge_index[0].astype(jnp.int32), (0, EP - E),
                  constant_values=N).reshape(NW, NCH, CH)
    dst = jnp.pad(edge_index[1].astype(jnp.int32), (0, EP - E),
                  constant_values=N).reshape(NW, NCH, CH)
    feat3 = jnp.pad(feat, ((0, NP - N), (0, 0))).reshape(NP // 128, 128, D)

    do_p, di_p = _sc_degrees(src, dst)
    do3 = do_p.reshape(NC, NP // 128, 128)
    di3 = di_p.reshape(NC, NP // 128, 128)

    h1 = _tc1(feat3, W1, do3)
    agg1 = _sc_aggregate(h1, src, dst)
    h2 = _tc2(agg1.reshape(NC, NP // 128, 128, D), di3, do3, b1, W2)
    agg2 = _sc_aggregate(h2, src, dst)
    out = _tc3(agg2.reshape(NC, NP // 128, 128, D), di3, b2, feat3)
    return out[:N]


# speedup vs baseline: 1.0490x; 1.0490x over previous
import functools

import jax
import jax.numpy as jnp
from jax import lax
from jax.experimental import pallas as pl
from jax.experimental.pallas import tpu as pltpu
from jax.experimental.pallas import tpu_sc as plsc

N = 10000
NP = 10240
D = 128
E = 320000
NC = 2
NS = 16
NW = NC * NS
CH = 80
NCH = 128
EP = NW * NCH * CH
RPT = NP // NS

_mesh = plsc.VectorSubcoreMesh(core_axis_name="c", subcore_axis_name="s")


@functools.partial(
    pl.kernel,
    out_type=(
        jax.ShapeDtypeStruct((NC * NP,), jnp.float32),
        jax.ShapeDtypeStruct((NC * NP,), jnp.float32),
    ),
    mesh=_mesh,
    scratch_types=[
        pltpu.VMEM((NCH, CH), jnp.int32),
        pltpu.VMEM((NCH, CH), jnp.int32),
        pltpu.VMEM((CH,), jnp.float32),
        pltpu.VMEM((RPT,), jnp.float32),
        pltpu.VMEM_SHARED((NP,), jnp.float32),
        pltpu.VMEM_SHARED((NP,), jnp.float32),
    ],
)
def _sc_degrees(src_hbm, dst_hbm, do_hbm, di_hbm,
                idx_s, idx_d, ones_v, zero_v, acc_o, acc_i):
    c = lax.axis_index("c")
    s = lax.axis_index("s")
    t = c * NS + s

    for i in range(CH // 16):
        ones_v[pl.ds(i * 16, 16)] = jnp.full((16,), 1.0, jnp.float32)
    for i in range(RPT // 16):
        zero_v[pl.ds(i * 16, 16)] = jnp.zeros((16,), jnp.float32)

    pltpu.sync_copy(zero_v, acc_o.at[pl.ds(s * RPT, RPT)])
    pltpu.sync_copy(zero_v, acc_i.at[pl.ds(s * RPT, RPT)])
    pltpu.sync_copy(src_hbm.at[t], idx_s)
    pltpu.sync_copy(dst_hbm.at[t], idx_d)
    plsc.subcore_barrier()

    def body(j, carry):
        pltpu.sync_copy(ones_v, acc_o.at[idx_s.at[j]], add=True)
        pltpu.sync_copy(ones_v, acc_i.at[idx_d.at[j]], add=True)
        return carry

    lax.fori_loop(0, NCH, body, 0)
    plsc.subcore_barrier()

    pltpu.sync_copy(acc_o.at[pl.ds(s * RPT, RPT)], do_hbm.at[pl.ds(c * NP + s * RPT, RPT)])
    pltpu.sync_copy(acc_i.at[pl.ds(s * RPT, RPT)], di_hbm.at[pl.ds(c * NP + s * RPT, RPT)])


@functools.partial(
    pl.kernel,
    out_type=jax.ShapeDtypeStruct((NC, NP, D), jnp.float32),
    mesh=_mesh,
    scratch_types=[
        pltpu.VMEM((NCH, CH), jnp.int32),
        pltpu.VMEM((NCH, CH), jnp.int32),
        pltpu.VMEM((CH, D), jnp.float32),
        pltpu.VMEM((16, D), jnp.float32),
        pltpu.VMEM_SHARED((NP, D), jnp.float32),
    ],
)
def _sc_aggregate(h_hbm, src_hbm, dst_hbm, out_hbm,
                  idx_s, idx_d, rows_v, zero_v, acc):
    c = lax.axis_index("c")
    s = lax.axis_index("s")
    t = c * NS + s

    for i in range(16):
        for cb in range(D // 16):
            zero_v[i, pl.ds(cb * 16, 16)] = jnp.zeros((16,), jnp.float32)

    def zbody(k, carry):
        pltpu.sync_copy(zero_v, acc.at[pl.ds(s * RPT + k * 16, 16)])
        return carry

    lax.fori_loop(0, RPT // 16, zbody, 0)
    pltpu.sync_copy(src_hbm.at[t], idx_s)
    pltpu.sync_copy(dst_hbm.at[t], idx_d)
    plsc.subcore_barrier()

    def body(j, carry):
        pltpu.sync_copy(h_hbm.at[idx_s.at[j]], rows_v)
        pltpu.sync_copy(rows_v, acc.at[idx_d.at[j]], add=True)
        return carry

    lax.fori_loop(0, NCH, body, 0)
    plsc.subcore_barrier()

    pltpu.sync_copy(acc.at[pl.ds(s * RPT, RPT)], out_hbm.at[c, pl.ds(s * RPT, RPT)])


RB = 2048
GRID = NP // RB
RT = RB // 128


def _norm(p):
    return lax.rsqrt(jnp.maximum(p[0] + p[1], 1.0))


def _tc1_body(feat_ref, w_ref, do_ref, out_ref):
    ns = _norm(do_ref[...])
    scaled = feat_ref[...] * ns[:, :, None]
    out_ref[...] = jnp.dot(scaled.reshape(RB, D), w_ref[...],
                           preferred_element_type=jnp.float32)


def _tc1(feat3, W1, do3):
    return pl.pallas_call(
        _tc1_body,
        grid=(GRID,),
        in_specs=[
            pl.BlockSpec((RT, 128, D), lambda i: (i, 0, 0)),
            pl.BlockSpec((D, D), lambda i: (0, 0)),
            pl.BlockSpec((NC, RT, 128), lambda i: (0, i, 0)),
        ],
        out_specs=pl.BlockSpec((RB, D), lambda i: (i, 0)),
        out_shape=jax.ShapeDtypeStruct((NP, D), jnp.float32),
    )(feat3, W1, do3)


def _tc2_body(agg_ref, di_ref, do_ref, b_ref, w_ref, out_ref):
    nd = _norm(di_ref[...])
    ns = _norm(do_ref[...])
    a = agg_ref[0] + agg_ref[1]
    h = a * nd[:, :, None] + b_ref[...]
    m = jnp.dot(h.reshape(RB, D), w_ref[...], preferred_element_type=jnp.float32)
    out_ref[...] = (m.reshape(RT, 128, D) * ns[:, :, None]).reshape(RB, D)


def _tc2(agg4, di3, do3, b1, W2):
    return pl.pallas_call(
        _tc2_body,
        grid=(GRID,),
        in_specs=[
            pl.BlockSpec((NC, RT, 128, D), lambda i: (0, i, 0, 0)),
            pl.BlockSpec((NC, RT, 128), lambda i: (0, i, 0)),
            pl.BlockSpec((NC, RT, 128), lambda i: (0, i, 0)),
            pl.BlockSpec((1, D), lambda i: (0, 0)),
            pl.BlockSpec((D, D), lambda i: (0, 0)),
        ],
        out_specs=pl.BlockSpec((RB, D), lambda i: (i, 0)),
        out_shape=jax.ShapeDtypeStruct((NP, D), jnp.float32),
    )(agg4, di3, do3, b1.reshape(1, D), W2)


def _tc3_body(agg_ref, di_ref, b_ref, feat_ref, out_ref):
    nd = _norm(di_ref[...])
    a = agg_ref[0] + agg_ref[1]
    out3 = a * nd[:, :, None] + b_ref[...] + feat_ref[...]
    out_ref[...] = out3.reshape(RB, D)


def _tc3(agg4, di3, b2, feat3):
    return pl.pallas_call(
        _tc3_body,
        grid=(GRID,),
        in_specs=[
            pl.BlockSpec((NC, RT, 128, D), lambda i: (0, i, 0, 0)),
            pl.BlockSpec((NC, RT, 128), lambda i: (0, i, 0)),
            pl.BlockSpec((1, D), lambda i: (0, 0)),
            pl.BlockSpec((RT, 128, D), lambda i: (i, 0, 0)),
        ],
        out_specs=pl.BlockSpec((RB, D), lambda i: (i, 0)),
        out_shape=jax.ShapeDtypeStruct((NP, D), jnp.float32),
    )(agg4, di3, b2.reshape(1, D), feat3)


def kernel(feat, edge_index, W1, b1, W2, b2):
    src = jnp.pad(edge_index[0].astype(jnp.int32), (0, EP - E),
                  constant_values=N).reshape(NW, NCH, CH)
    dst = jnp.pad(edge_index[1].astype(jnp.int32), (0, EP - E),
                  constant_values=N).reshape(NW, NCH, CH)
    feat3 = jnp.pad(feat, ((0, NP - N), (0, 0))).reshape(NP // 128, 128, D)

    do_p, di_p = _sc_degrees(src, dst)
    do3 = do_p.reshape(NC, NP // 128, 128)
    di3 = di_p.reshape(NC, NP // 128, 128)

    h1 = _tc1(feat3, W1, do3)
    agg1 = _sc_aggregate(h1, src, dst)
    h2 = _tc2(agg1.reshape(NC, NP // 128, 128, D), di3, do3, b1, W2)
    agg2 = _sc_aggregate(h2, src, dst)
    out = _tc3(agg2.reshape(NC, NP // 128, 128, D), di3, b2, feat3)
    return out[:N]


# speedup vs baseline: 3.2894x; 3.1358x over previous
import functools

import jax
import jax.numpy as jnp
from jax import lax
from jax.experimental import pallas as pl
from jax.experimental.pallas import tpu as pltpu
from jax.experimental.pallas import tpu_sc as plsc

N = 10000
NP = 10240
D = 128
DH = D // 2
E = 320000
NC = 2
NS = 16
NW = NC * NS
CH = 128
NCH = 160
NCHD = NCH // NC
EP = NS * NCH * CH
RPT = NP // NS

_mesh = plsc.VectorSubcoreMesh(core_axis_name="c", subcore_axis_name="s")


@functools.partial(
    pl.kernel,
    out_type=(
        jax.ShapeDtypeStruct((NC * NP,), jnp.float32),
        jax.ShapeDtypeStruct((NC * NP,), jnp.float32),
    ),
    mesh=_mesh,
    compiler_params=pltpu.CompilerParams(use_tc_tiling_on_sc=False),
    scratch_types=[
        pltpu.VMEM((NCHD, CH), jnp.int32),
        pltpu.VMEM((NCHD, CH), jnp.int32),
        pltpu.VMEM((CH,), jnp.float32),
        pltpu.VMEM((RPT,), jnp.float32),
        pltpu.VMEM_SHARED((NP,), jnp.float32),
        pltpu.VMEM_SHARED((NP,), jnp.float32),
    ],
)
def _sc_degrees(src_hbm, dst_hbm, do_hbm, di_hbm,
                idx_s, idx_d, ones_v, zero_v, acc_o, acc_i):
    c = lax.axis_index("c")
    s = lax.axis_index("s")

    for i in range(CH // 16):
        ones_v[pl.ds(i * 16, 16)] = jnp.full((16,), 1.0, jnp.float32)
    for i in range(RPT // 16):
        zero_v[pl.ds(i * 16, 16)] = jnp.zeros((16,), jnp.float32)

    pltpu.sync_copy(zero_v, acc_o.at[pl.ds(s * RPT, RPT)])
    pltpu.sync_copy(zero_v, acc_i.at[pl.ds(s * RPT, RPT)])
    pltpu.sync_copy(src_hbm.at[s, pl.ds(c * NCHD, NCHD)], idx_s)
    pltpu.sync_copy(dst_hbm.at[s, pl.ds(c * NCHD, NCHD)], idx_d)
    plsc.subcore_barrier()

    def body(j, carry):
        pltpu.sync_copy(ones_v, acc_o.at[idx_s.at[j]], add=True)
        pltpu.sync_copy(ones_v, acc_i.at[idx_d.at[j]], add=True)
        return carry

    lax.fori_loop(0, NCHD, body, 0)
    plsc.subcore_barrier()

    pltpu.sync_copy(acc_o.at[pl.ds(s * RPT, RPT)], do_hbm.at[pl.ds(c * NP + s * RPT, RPT)])
    pltpu.sync_copy(acc_i.at[pl.ds(s * RPT, RPT)], di_hbm.at[pl.ds(c * NP + s * RPT, RPT)])


@functools.partial(
    pl.kernel,
    out_type=jax.ShapeDtypeStruct((NC, NP, DH), jnp.float32),
    mesh=_mesh,
    compiler_params=pltpu.CompilerParams(use_tc_tiling_on_sc=False),
    scratch_types=[
        pltpu.VMEM((NCH, CH), jnp.int32),
        pltpu.VMEM((NCH, CH), jnp.int32),
        pltpu.VMEM((4, CH, DH), jnp.float32),
        pltpu.VMEM((16, DH), jnp.float32),
        pltpu.VMEM_SHARED((NP, DH), jnp.float32),
        pltpu.SemaphoreType.DMA,
        pltpu.SemaphoreType.DMA,
        pltpu.SemaphoreType.DMA,
        pltpu.SemaphoreType.DMA,
    ],
)
def _sc_aggregate(h2s_hbm, src_hbm, dst_hbm, out_hbm,
                  idx_s, idx_d, rows_v, zero_v, acc,
                  sem0, sem1, sem2, sem3):
    c = lax.axis_index("c")
    s = lax.axis_index("s")
    sems = (sem0, sem1, sem2, sem3)

    def fire(slot, chunk):
        pltpu.async_copy(h2s_hbm.at[c].at[idx_s.at[chunk]], rows_v.at[slot],
                         sems[slot])

    def drain_scatter(slot, chunk):
        pltpu.make_async_copy(
            h2s_hbm.at[c].at[idx_s.at[0]], rows_v.at[slot], sems[slot]).wait()
        pltpu.sync_copy(rows_v.at[slot], acc.at[idx_d.at[chunk]], add=True)

    for i in range(16):
        for cb in range(DH // 16):
            zero_v[i, pl.ds(cb * 16, 16)] = jnp.zeros((16,), jnp.float32)

    def zbody(k, carry):
        pltpu.sync_copy(zero_v, acc.at[pl.ds(s * RPT + k * 16, 16)])
        return carry

    lax.fori_loop(0, RPT // 16, zbody, 0)
    pltpu.sync_copy(src_hbm.at[s], idx_s)
    pltpu.sync_copy(dst_hbm.at[s], idx_d)
    plsc.subcore_barrier()

    fire(0, 0)
    fire(1, 1)

    def body(g, carry):
        base = g * 4
        fire(2, base + 2)
        fire(3, base + 3)
        drain_scatter(0, base)
        drain_scatter(1, base + 1)
        fire(0, base + 4)
        fire(1, base + 5)
        drain_scatter(2, base + 2)
        drain_scatter(3, base + 3)
        return carry

    lax.fori_loop(0, NCH // 4 - 1, body, 0)
    base = NCH - 4
    fire(2, base + 2)
    fire(3, base + 3)
    drain_scatter(0, base)
    drain_scatter(1, base + 1)
    drain_scatter(2, base + 2)
    drain_scatter(3, base + 3)
    plsc.subcore_barrier()

    pltpu.sync_copy(acc.at[pl.ds(s * RPT, RPT)], out_hbm.at[c, pl.ds(s * RPT, RPT)])


RB = 2048
GRID = NP // RB
RT = RB // 128


def _norm(p):
    return lax.rsqrt(jnp.maximum(p[0] + p[1], 1.0))


def _split(m):
    return jnp.stack([m[:, :DH], m[:, DH:]], axis=0)


def _tc1_body(feat_ref, w_ref, do_ref, out_ref):
    ns = _norm(do_ref[...])
    scaled = feat_ref[...] * ns[:, :, None]
    m = jnp.dot(scaled.reshape(RB, D), w_ref[...],
                preferred_element_type=jnp.float32)
    out_ref[...] = _split(m)


def _tc1(feat3, W1, do3):
    return pl.pallas_call(
        _tc1_body,
        grid=(GRID,),
        in_specs=[
            pl.BlockSpec((RT, 128, D), lambda i: (i, 0, 0)),
            pl.BlockSpec((D, D), lambda i: (0, 0)),
            pl.BlockSpec((NC, RT, 128), lambda i: (0, i, 0)),
        ],
        out_specs=pl.BlockSpec((NC, RB, DH), lambda i: (0, i, 0)),
        out_shape=jax.ShapeDtypeStruct((NC, NP, DH), jnp.float32),
    )(feat3, W1, do3)


def _tc2_body(agg_ref, di_ref, do_ref, b_ref, w_ref, out_ref):
    nd = _norm(di_ref[...])
    ns = _norm(do_ref[...])
    a = jnp.concatenate([agg_ref[0], agg_ref[1]], axis=-1)
    h = a * nd[:, :, None] + b_ref[...]
    m = jnp.dot(h.reshape(RB, D), w_ref[...], preferred_element_type=jnp.float32)
    m = (m.reshape(RT, 128, D) * ns[:, :, None]).reshape(RB, D)
    out_ref[...] = _split(m)


def _tc2(agg4, di3, do3, b1, W2):
    return pl.pallas_call(
        _tc2_body,
        grid=(GRID,),
        in_specs=[
            pl.BlockSpec((NC, RT, 128, DH), lambda i: (0, i, 0, 0)),
            pl.BlockSpec((NC, RT, 128), lambda i: (0, i, 0)),
            pl.BlockSpec((NC, RT, 128), lambda i: (0, i, 0)),
            pl.BlockSpec((1, D), lambda i: (0, 0)),
            pl.BlockSpec((D, D), lambda i: (0, 0)),
        ],
        out_specs=pl.BlockSpec((NC, RB, DH), lambda i: (0, i, 0)),
        out_shape=jax.ShapeDtypeStruct((NC, NP, DH), jnp.float32),
    )(agg4, di3, do3, b1.reshape(1, D), W2)


def _tc3_body(agg_ref, di_ref, b_ref, feat_ref, out_ref):
    nd = _norm(di_ref[...])
    a = jnp.concatenate([agg_ref[0], agg_ref[1]], axis=-1)
    out3 = a * nd[:, :, None] + b_ref[...] + feat_ref[...]
    out_ref[...] = out3.reshape(RB, D)


def _tc3(agg4, di3, b2, feat3):
    return pl.pallas_call(
        _tc3_body,
        grid=(GRID,),
        in_specs=[
            pl.BlockSpec((NC, RT, 128, DH), lambda i: (0, i, 0, 0)),
            pl.BlockSpec((NC, RT, 128), lambda i: (0, i, 0)),
            pl.BlockSpec((1, D), lambda i: (0, 0)),
            pl.BlockSpec((RT, 128, D), lambda i: (i, 0, 0)),
        ],
        out_specs=pl.BlockSpec((RB, D), lambda i: (i, 0)),
        out_shape=jax.ShapeDtypeStruct((NP, D), jnp.float32),
    )(agg4, di3, b2.reshape(1, D), feat3)


def kernel(feat, edge_index, W1, b1, W2, b2):
    pad_idx = N + (jnp.arange(EP - E, dtype=jnp.int32) % (NP - N))
    src = jnp.concatenate(
        [edge_index[0].astype(jnp.int32), pad_idx]).reshape(NS, NCH, CH)
    dst = jnp.concatenate(
        [edge_index[1].astype(jnp.int32), pad_idx]).reshape(NS, NCH, CH)
    feat3 = jnp.pad(feat, ((0, NP - N), (0, 0))).reshape(NP // 128, 128, D)

    do_p, di_p = _sc_degrees(src, dst)
    do3 = do_p.reshape(NC, NP // 128, 128)
    di3 = di_p.reshape(NC, NP // 128, 128)

    h1 = _tc1(feat3, W1, do3)
    agg1 = _sc_aggregate(h1, src, dst)
    h2 = _tc2(agg1.reshape(NC, NP // 128, 128, DH), di3, do3, b1, W2)
    agg2 = _sc_aggregate(h2, src, dst)
    out = _tc3(agg2.reshape(NC, NP // 128, 128, DH), di3, b2, feat3)
    return out[:N]
